# Initial kernel scaffold; baseline (speedup 1.0000x reference)
import functools

import jax
import jax.numpy as jnp
from jax import lax
from jax.experimental import pallas as pl
from jax.experimental.pallas import tpu as pltpu
from jax.experimental.pallas import tpu_sc as plsc

VOCAB = 100000
HIDDEN = 768
MAX_POS = 512
EPS = 1e-12

GATHER_WINDOW = 64
TOKEN_BLOCK = 512


def _sc_gather(word_emb, flat_ids, n_tokens):
    mesh = plsc.VectorSubcoreMesh(core_axis_name="c", subcore_axis_name="s")
    ids_2d = flat_ids.reshape(1, n_tokens)

    @functools.partial(
        pl.kernel,
        out_type=jax.ShapeDtypeStruct((n_tokens, HIDDEN), jnp.float32),
        mesh=mesh,
    )
    def gather_kernel(table_hbm, idx_hbm, out_hbm):
        def body(idx_vmem, out_vmem):
            pltpu.sync_copy(table_hbm.at[idx_vmem.at[0]], out_vmem)

        pltpu.emit_pipeline(
            body,
            grid=(n_tokens // GATHER_WINDOW,),
            in_specs=[
                pl.BlockSpec((1, GATHER_WINDOW), index_map=lambda i: (0, i)),
            ],
            out_specs=[
                pl.BlockSpec((GATHER_WINDOW, HIDDEN), index_map=lambda i: (i, 0)),
            ],
            core_axis_name=("c", "s"),
            dimension_semantics=(pltpu.PARALLEL,),
        )(idx_hbm, out_hbm)

    return gather_kernel(word_emb, ids_2d)


def _tc_body(w_ref, pid_ref, tid_ref, pos_ref, type_ref, gamma_ref, beta_ref,
             out_ref):
    pid = pid_ref[0]
    tid = tid_ref[0]
    pos_iota = lax.broadcasted_iota(jnp.int32, (TOKEN_BLOCK, MAX_POS), 1)
    onehot = (pid == pos_iota).astype(jnp.bfloat16)
    pos_rows = jnp.dot(onehot, pos_ref[...],
                       preferred_element_type=jnp.float32)
    type_rows = jnp.where(tid == 1, type_ref[1, :][None, :],
                          type_ref[0, :][None, :])
    x = w_ref[...] + pos_rows + type_rows
    mean = jnp.mean(x, axis=1, keepdims=True)
    xc = x - mean
    var = jnp.mean(xc * xc, axis=1, keepdims=True)
    xhat = xc * lax.rsqrt(var + EPS)
    out_ref[...] = xhat * gamma_ref[...] + beta_ref[...]


def _tc_fuse(w_rows, pos_ids, type_ids, pos_emb, type_emb, gamma, beta,
             n_tokens):
    n_blocks = n_tokens // TOKEN_BLOCK
    pid3 = pos_ids.reshape(n_blocks, TOKEN_BLOCK, 1)
    tid3 = type_ids.reshape(n_blocks, TOKEN_BLOCK, 1)
    pos_bf16 = pos_emb.astype(jnp.bfloat16)

    return pl.pallas_call(
        _tc_body,
        grid=(n_blocks,),
        in_specs=[
            pl.BlockSpec((TOKEN_BLOCK, HIDDEN), lambda i: (i, 0)),
            pl.BlockSpec((1, TOKEN_BLOCK, 1), lambda i: (i, 0, 0)),
            pl.BlockSpec((1, TOKEN_BLOCK, 1), lambda i: (i, 0, 0)),
            pl.BlockSpec((MAX_POS, HIDDEN), lambda i: (0, 0)),
            pl.BlockSpec((2, HIDDEN), lambda i: (0, 0)),
            pl.BlockSpec((1, HIDDEN), lambda i: (0, 0)),
            pl.BlockSpec((1, HIDDEN), lambda i: (0, 0)),
        ],
        out_specs=pl.BlockSpec((TOKEN_BLOCK, HIDDEN), lambda i: (i, 0)),
        out_shape=jax.ShapeDtypeStruct((n_tokens, HIDDEN), jnp.float32),
    )(w_rows, pid3, tid3, pos_bf16, type_emb, gamma.reshape(1, HIDDEN),
      beta.reshape(1, HIDDEN))


@jax.jit
def kernel(input_ids, position_ids, token_type_ids, word_emb, pos_emb,
           type_emb, gamma, beta):
    b, l = input_ids.shape
    n_tokens = b * l
    w_rows = _sc_gather(word_emb, input_ids.reshape(-1), n_tokens)
    out = _tc_fuse(w_rows, position_ids.reshape(-1), token_type_ids.reshape(-1),
                   pos_emb, type_emb, gamma, beta, n_tokens)
    return out.reshape(b, l, HIDDEN)


# speedup vs baseline: 1.3652x; 1.3652x over previous
import functools

import jax
import jax.numpy as jnp
from jax import lax
from jax.experimental import pallas as pl
from jax.experimental.pallas import tpu as pltpu
from jax.experimental.pallas import tpu_sc as plsc

VOCAB = 100000
HIDDEN = 768
MAX_POS = 512
EPS = 1e-12

GATHER_WINDOW = 128
HALF = HIDDEN // 2
TOKEN_BLOCK = 512


def _sc_gather(word_emb, flat_ids, n_tokens):
    n_half = 2 * n_tokens
    table_half = word_emb.reshape(2 * VOCAB, HALF)
    half_ids = (flat_ids[:, None] * 2
                + jnp.arange(2, dtype=jnp.int32)[None, :]).reshape(1, n_half)
    mesh = plsc.VectorSubcoreMesh(core_axis_name="c", subcore_axis_name="s")

    @functools.partial(
        pl.kernel,
        out_type=jax.ShapeDtypeStruct((n_half, HALF), jnp.float32),
        mesh=mesh,
    )
    def gather_kernel(table_hbm, idx_hbm, out_hbm):
        def body(idx_vmem, out_vmem):
            pltpu.sync_copy(table_hbm.at[idx_vmem.at[0]], out_vmem)

        pltpu.emit_pipeline(
            body,
            grid=(n_half // GATHER_WINDOW,),
            in_specs=[
                pl.BlockSpec((1, GATHER_WINDOW), index_map=lambda i: (0, i)),
            ],
            out_specs=[
                pl.BlockSpec((GATHER_WINDOW, HALF), index_map=lambda i: (i, 0)),
            ],
            core_axis_name=("c", "s"),
            dimension_semantics=(pltpu.PARALLEL,),
        )(idx_hbm, out_hbm)

    return gather_kernel(table_half, half_ids).reshape(n_tokens, HIDDEN)


def _tc_body(w_ref, pid_ref, tid_ref, pos_ref, type_ref, gamma_ref, beta_ref,
             out_ref):
    pid = pid_ref[0]
    tid = tid_ref[0]
    pos_iota = lax.broadcasted_iota(jnp.int32, (TOKEN_BLOCK, MAX_POS), 1)
    onehot = (pid == pos_iota).astype(jnp.bfloat16)
    pos_rows = jnp.dot(onehot, pos_ref[...],
                       preferred_element_type=jnp.float32)
    type_rows = jnp.where(tid == 1, type_ref[1, :][None, :],
                          type_ref[0, :][None, :])
    x = w_ref[...] + pos_rows + type_rows
    mean = jnp.mean(x, axis=1, keepdims=True)
    xc = x - mean
    var = jnp.mean(xc * xc, axis=1, keepdims=True)
    xhat = xc * lax.rsqrt(var + EPS)
    out_ref[...] = xhat * gamma_ref[...] + beta_ref[...]


def _tc_fuse(w_rows, pos_ids, type_ids, pos_emb, type_emb, gamma, beta,
             n_tokens):
    n_blocks = n_tokens // TOKEN_BLOCK
    pid3 = pos_ids.reshape(n_blocks, TOKEN_BLOCK, 1)
    tid3 = type_ids.reshape(n_blocks, TOKEN_BLOCK, 1)
    pos_bf16 = pos_emb.astype(jnp.bfloat16)

    return pl.pallas_call(
        _tc_body,
        grid=(n_blocks,),
        in_specs=[
            pl.BlockSpec((TOKEN_BLOCK, HIDDEN), lambda i: (i, 0)),
            pl.BlockSpec((1, TOKEN_BLOCK, 1), lambda i: (i, 0, 0)),
            pl.BlockSpec((1, TOKEN_BLOCK, 1), lambda i: (i, 0, 0)),
            pl.BlockSpec((MAX_POS, HIDDEN), lambda i: (0, 0)),
            pl.BlockSpec((2, HIDDEN), lambda i: (0, 0)),
            pl.BlockSpec((1, HIDDEN), lambda i: (0, 0)),
            pl.BlockSpec((1, HIDDEN), lambda i: (0, 0)),
        ],
        out_specs=pl.BlockSpec((TOKEN_BLOCK, HIDDEN), lambda i: (i, 0)),
        out_shape=jax.ShapeDtypeStruct((n_tokens, HIDDEN), jnp.float32),
    )(w_rows, pid3, tid3, pos_bf16, type_emb, gamma.reshape(1, HIDDEN),
      beta.reshape(1, HIDDEN))


@jax.jit
def kernel(input_ids, position_ids, token_type_ids, word_emb, pos_emb,
           type_emb, gamma, beta):
    b, l = input_ids.shape
    n_tokens = b * l
    w_rows = _sc_gather(word_emb, input_ids.reshape(-1), n_tokens)
    out = _tc_fuse(w_rows, position_ids.reshape(-1), token_type_ids.reshape(-1),
                   pos_emb, type_emb, gamma, beta, n_tokens)
    return out.reshape(b, l, HIDDEN)


# speedup vs baseline: 1.4181x; 1.0387x over previous
import functools

import jax
import jax.numpy as jnp
from jax import lax
from jax.experimental import pallas as pl
from jax.experimental.pallas import tpu as pltpu
from jax.experimental.pallas import tpu_sc as plsc

VOCAB = 100000
HIDDEN = 768
MAX_POS = 512
EPS = 1e-12

GATHER_WINDOW = 128
HALF = HIDDEN // 2
TOKEN_BLOCK = 512


def _sc_gather(word_emb, flat_ids, n_tokens):
    n_half = 2 * n_tokens
    table_half = word_emb.reshape(2 * VOCAB, HALF)
    half_ids = (flat_ids[:, None] * 2
                + jnp.arange(2, dtype=jnp.int32)[None, :]).reshape(1, n_half)
    mesh = plsc.VectorSubcoreMesh(core_axis_name="c", subcore_axis_name="s")

    @functools.partial(
        pl.kernel,
        out_type=jax.ShapeDtypeStruct((n_half, HALF), jnp.float32),
        mesh=mesh,
    )
    def gather_kernel(table_hbm, idx_hbm, out_hbm):
        def body(idx_vmem, out_vmem):
            pltpu.sync_copy(table_hbm.at[idx_vmem.at[0]], out_vmem)

        pltpu.emit_pipeline(
            body,
            grid=(n_half // GATHER_WINDOW,),
            in_specs=[
                pl.BlockSpec((1, GATHER_WINDOW), index_map=lambda i: (0, i)),
            ],
            out_specs=[
                pl.BlockSpec((GATHER_WINDOW, HALF), index_map=lambda i: (i, 0)),
            ],
            core_axis_name=("c", "s"),
            dimension_semantics=(pltpu.PARALLEL,),
        )(idx_hbm, out_hbm)

    return gather_kernel(table_half, half_ids).reshape(n_tokens, HIDDEN)


EXT = MAX_POS + 8


def _tc_body(w_ref, pid_ref, tid_ref, ext_ref, gamma_ref, beta_ref, out_ref):
    pid = pid_ref[0]
    tid = tid_ref[0]
    iota = lax.broadcasted_iota(jnp.int32, (EXT, TOKEN_BLOCK), 0)
    oh = ((iota == pid) | (iota - MAX_POS == tid)).astype(jnp.bfloat16)
    pt = lax.dot_general(oh, ext_ref[...], (((0,), (0,)), ((), ())),
                         preferred_element_type=jnp.float32)
    x = w_ref[...] + pt
    s1 = jnp.sum(x, axis=1, keepdims=True)
    s2 = jnp.sum(x * x, axis=1, keepdims=True)
    m = s1 * (1.0 / HIDDEN)
    var = s2 * (1.0 / HIDDEN) - m * m
    r = lax.rsqrt(var + EPS)
    out_ref[...] = (x - m) * r * gamma_ref[...] + beta_ref[...]


def _tc_fuse(w_rows, pos_ids, type_ids, pos_emb, type_emb, gamma, beta,
             n_tokens):
    n_blocks = n_tokens // TOKEN_BLOCK
    pid3 = pos_ids.reshape(n_blocks, 1, TOKEN_BLOCK)
    tid3 = type_ids.reshape(n_blocks, 1, TOKEN_BLOCK)
    ext_table = jnp.concatenate(
        [pos_emb, type_emb, jnp.zeros((EXT - MAX_POS - 2, HIDDEN),
                                      jnp.float32)],
        axis=0).astype(jnp.bfloat16)

    return pl.pallas_call(
        _tc_body,
        grid=(n_blocks,),
        in_specs=[
            pl.BlockSpec((TOKEN_BLOCK, HIDDEN), lambda i: (i, 0)),
            pl.BlockSpec((1, 1, TOKEN_BLOCK), lambda i: (i, 0, 0)),
            pl.BlockSpec((1, 1, TOKEN_BLOCK), lambda i: (i, 0, 0)),
            pl.BlockSpec((EXT, HIDDEN), lambda i: (0, 0)),
            pl.BlockSpec((1, HIDDEN), lambda i: (0, 0)),
            pl.BlockSpec((1, HIDDEN), lambda i: (0, 0)),
        ],
        out_specs=pl.BlockSpec((TOKEN_BLOCK, HIDDEN), lambda i: (i, 0)),
        out_shape=jax.ShapeDtypeStruct((n_tokens, HIDDEN), jnp.float32),
    )(w_rows, pid3, tid3, ext_table, gamma.reshape(1, HIDDEN),
      beta.reshape(1, HIDDEN))


@jax.jit
def kernel(input_ids, position_ids, token_type_ids, word_emb, pos_emb,
           type_emb, gamma, beta):
    b, l = input_ids.shape
    n_tokens = b * l
    w_rows = _sc_gather(word_emb, input_ids.reshape(-1), n_tokens)
    out = _tc_fuse(w_rows, position_ids.reshape(-1), token_type_ids.reshape(-1),
                   pos_emb, type_emb, gamma, beta, n_tokens)
    return out.reshape(b, l, HIDDEN)


# speedup vs baseline: 1.4971x; 1.0557x over previous
import functools

import jax
import jax.numpy as jnp
from jax import lax
from jax.experimental import pallas as pl
from jax.experimental.pallas import tpu as pltpu
from jax.experimental.pallas import tpu_sc as plsc

VOCAB = 100000
HIDDEN = 768
MAX_POS = 512
EPS = 1e-12

GATHER_WINDOW = 128
HALF = HIDDEN // 2
TOKEN_BLOCK = 1024


def _sc_gather(word_emb, flat_ids, n_tokens):
    n_half = 2 * n_tokens
    table_half = word_emb.reshape(2 * VOCAB, HALF)
    half_ids = (flat_ids[:, None] * 2
                + jnp.arange(2, dtype=jnp.int32)[None, :]).reshape(1, n_half)
    mesh = plsc.VectorSubcoreMesh(core_axis_name="c", subcore_axis_name="s")

    @functools.partial(
        pl.kernel,
        out_type=jax.ShapeDtypeStruct((n_half, HALF), jnp.float32),
        mesh=mesh,
    )
    def gather_kernel(table_hbm, idx_hbm, out_hbm):
        def body(idx_vmem, out_vmem):
            pltpu.sync_copy(table_hbm.at[idx_vmem.at[0]], out_vmem)

        pltpu.emit_pipeline(
            body,
            grid=(n_half // GATHER_WINDOW,),
            in_specs=[
                pl.BlockSpec((1, GATHER_WINDOW), index_map=lambda i: (0, i)),
            ],
            out_specs=[
                pl.BlockSpec((GATHER_WINDOW, HALF), index_map=lambda i: (i, 0)),
            ],
            core_axis_name=("c", "s"),
            dimension_semantics=(pltpu.PARALLEL,),
        )(idx_hbm, out_hbm)

    return gather_kernel(table_half, half_ids).reshape(n_tokens, HIDDEN)


EXT = MAX_POS + 8


def _tc_body(w_ref, pid_ref, tid_ref, ext_ref, gamma_ref, beta_ref, out_ref):
    pid = pid_ref[0]
    tid = tid_ref[0]
    iota = lax.broadcasted_iota(jnp.int32, (EXT, TOKEN_BLOCK), 0)
    oh = ((iota == pid) | (iota - MAX_POS == tid)).astype(jnp.bfloat16)
    pt = lax.dot_general(oh, ext_ref[...], (((0,), (0,)), ((), ())),
                         preferred_element_type=jnp.float32)
    x = w_ref[...] + pt
    s1 = jnp.sum(x, axis=1, keepdims=True)
    s2 = jnp.sum(x * x, axis=1, keepdims=True)
    m = s1 * (1.0 / HIDDEN)
    var = s2 * (1.0 / HIDDEN) - m * m
    r = lax.rsqrt(var + EPS)
    out_ref[...] = (x - m) * r * gamma_ref[...] + beta_ref[...]


def _tc_fuse(w_rows, pos_ids, type_ids, pos_emb, type_emb, gamma, beta,
             n_tokens):
    n_blocks = n_tokens // TOKEN_BLOCK
    pid3 = pos_ids.reshape(n_blocks, 1, TOKEN_BLOCK)
    tid3 = type_ids.reshape(n_blocks, 1, TOKEN_BLOCK)
    ext_table = jnp.concatenate(
        [pos_emb, type_emb, jnp.zeros((EXT - MAX_POS - 2, HIDDEN),
                                      jnp.float32)],
        axis=0).astype(jnp.bfloat16)

    return pl.pallas_call(
        _tc_body,
        grid=(n_blocks,),
        in_specs=[
            pl.BlockSpec((TOKEN_BLOCK, HIDDEN), lambda i: (i, 0)),
            pl.BlockSpec((1, 1, TOKEN_BLOCK), lambda i: (i, 0, 0)),
            pl.BlockSpec((1, 1, TOKEN_BLOCK), lambda i: (i, 0, 0)),
            pl.BlockSpec((EXT, HIDDEN), lambda i: (0, 0)),
            pl.BlockSpec((1, HIDDEN), lambda i: (0, 0)),
            pl.BlockSpec((1, HIDDEN), lambda i: (0, 0)),
        ],
        out_specs=pl.BlockSpec((TOKEN_BLOCK, HIDDEN), lambda i: (i, 0)),
        out_shape=jax.ShapeDtypeStruct((n_tokens, HIDDEN), jnp.float32),
    )(w_rows, pid3, tid3, ext_table, gamma.reshape(1, HIDDEN),
      beta.reshape(1, HIDDEN))


@jax.jit
def kernel(input_ids, position_ids, token_type_ids, word_emb, pos_emb,
           type_emb, gamma, beta):
    b, l = input_ids.shape
    n_tokens = b * l
    w_rows = _sc_gather(word_emb, input_ids.reshape(-1), n_tokens)
    out = _tc_fuse(w_rows, position_ids.reshape(-1), token_type_ids.reshape(-1),
                   pos_emb, type_emb, gamma, beta, n_tokens)
    return out.reshape(b, l, HIDDEN)


# speedup vs baseline: 3.2769x; 2.1888x over previous
import functools

import jax
import jax.numpy as jnp
from jax import lax
from jax.experimental import pallas as pl
from jax.experimental.pallas import tpu as pltpu
from jax.experimental.pallas import tpu_sc as plsc

VOCAB = 100000
HIDDEN = 768
MAX_POS = 512
EPS = 1e-12

TOKEN_BLOCK = 1024
NUM_WORKERS = 32
CHUNK = 64


def _sc_gather(word_emb, flat_ids, n_tokens):
    per_w = n_tokens // NUM_WORKERS
    n_chunks = per_w // CHUNK
    mesh = plsc.VectorSubcoreMesh(core_axis_name="c", subcore_axis_name="s")

    @functools.partial(
        pl.kernel,
        out_type=jax.ShapeDtypeStruct((n_tokens, HIDDEN), jnp.float32),
        mesh=mesh,
        scratch_types=[
            pltpu.VMEM((per_w,), jnp.int32),
            pltpu.VMEM((CHUNK, HIDDEN), jnp.float32),
            pltpu.VMEM((CHUNK, HIDDEN), jnp.float32),
            pltpu.SemaphoreType.DMA,
            pltpu.SemaphoreType.DMA,
            pltpu.SemaphoreType.DMA,
            pltpu.SemaphoreType.DMA,
        ],
    )
    def gather_kernel(table_hbm, idx_hbm, out_hbm, idx_v, buf_a, buf_b,
                      gs_a, gs_b, ws_a, ws_b):
        wid = lax.axis_index("s") * 2 + lax.axis_index("c")
        base = wid * per_w
        pltpu.sync_copy(idx_hbm.at[pl.ds(base, per_w)], idx_v)

        def gather(c, buf, sem):
            return pltpu.make_async_copy(
                table_hbm.at[idx_v.at[pl.ds(c * CHUNK, CHUNK)]], buf, sem)

        def wback(c, buf, sem):
            return pltpu.make_async_copy(
                buf, out_hbm.at[pl.ds(base + c * CHUNK, CHUNK)], sem)

        gather(0, buf_a, gs_a).start()
        gather(1, buf_b, gs_b).start()

        @pl.loop(0, n_chunks, step=2)
        def _(c):
            gather(c, buf_a, gs_a).wait()
            wback(c, buf_a, ws_a).start()
            gather(c + 1, buf_b, gs_b).wait()
            wback(c + 1, buf_b, ws_b).start()

            @pl.when(c + 2 < n_chunks)
            def _():
                wback(c, buf_a, ws_a).wait()
                gather(c + 2, buf_a, gs_a).start()
                wback(c + 1, buf_b, ws_b).wait()
                gather(c + 3, buf_b, gs_b).start()

        wback(n_chunks - 2, buf_a, ws_a).wait()
        wback(n_chunks - 1, buf_b, ws_b).wait()

    return gather_kernel(word_emb, flat_ids)


EXT = MAX_POS + 8


def _tc_body(w_ref, pid_ref, tid_ref, ext_ref, gamma_ref, beta_ref, out_ref):
    pid = pid_ref[0]
    tid = tid_ref[0]
    iota = lax.broadcasted_iota(jnp.int32, (EXT, TOKEN_BLOCK), 0)
    oh = ((iota == pid) | (iota - MAX_POS == tid)).astype(jnp.bfloat16)
    pt = lax.dot_general(oh, ext_ref[...], (((0,), (0,)), ((), ())),
                         preferred_element_type=jnp.float32)
    x = w_ref[...] + pt
    s1 = jnp.sum(x, axis=1, keepdims=True)
    s2 = jnp.sum(x * x, axis=1, keepdims=True)
    m = s1 * (1.0 / HIDDEN)
    var = s2 * (1.0 / HIDDEN) - m * m
    r = lax.rsqrt(var + EPS)
    out_ref[...] = (x - m) * r * gamma_ref[...] + beta_ref[...]


def _tc_fuse(w_rows, pos_ids, type_ids, pos_emb, type_emb, gamma, beta,
             n_tokens):
    n_blocks = n_tokens // TOKEN_BLOCK
    pid3 = pos_ids.reshape(n_blocks, 1, TOKEN_BLOCK)
    tid3 = type_ids.reshape(n_blocks, 1, TOKEN_BLOCK)
    ext_table = jnp.concatenate(
        [pos_emb, type_emb, jnp.zeros((EXT - MAX_POS - 2, HIDDEN),
                                      jnp.float32)],
        axis=0).astype(jnp.bfloat16)

    return pl.pallas_call(
        _tc_body,
        grid=(n_blocks,),
        in_specs=[
            pl.BlockSpec((TOKEN_BLOCK, HIDDEN), lambda i: (i, 0)),
            pl.BlockSpec((1, 1, TOKEN_BLOCK), lambda i: (i, 0, 0)),
            pl.BlockSpec((1, 1, TOKEN_BLOCK), lambda i: (i, 0, 0)),
            pl.BlockSpec((EXT, HIDDEN), lambda i: (0, 0)),
            pl.BlockSpec((1, HIDDEN), lambda i: (0, 0)),
            pl.BlockSpec((1, HIDDEN), lambda i: (0, 0)),
        ],
        out_specs=pl.BlockSpec((TOKEN_BLOCK, HIDDEN), lambda i: (i, 0)),
        out_shape=jax.ShapeDtypeStruct((n_tokens, HIDDEN), jnp.float32),
    )(w_rows, pid3, tid3, ext_table, gamma.reshape(1, HIDDEN),
      beta.reshape(1, HIDDEN))


@jax.jit
def kernel(input_ids, position_ids, token_type_ids, word_emb, pos_emb,
           type_emb, gamma, beta):
    b, l = input_ids.shape
    n_tokens = b * l
    w_rows = _sc_gather(word_emb, input_ids.reshape(-1), n_tokens)
    out = _tc_fuse(w_rows, position_ids.reshape(-1), token_type_ids.reshape(-1),
                   pos_emb, type_emb, gamma, beta, n_tokens)
    return out.reshape(b, l, HIDDEN)
